# baseline (speedup 1.0000x reference)
import functools

import jax
import jax.numpy as jnp
from jax import lax
from jax.experimental import pallas as pl
from jax.experimental.pallas import tpu as pltpu
from jax.experimental.pallas import tpu_sc as plsc

DIM = 256
N_EMBED = 8192
B = 16
HW = 1024
ROWS = B * HW
M_BLK = 256


N_CHUNK = 2816


def _argmin_body(x_ref, e_ref, out_ref, eb_ref, e2_ref):
    @pl.when(pl.program_id(0) == 0)
    def _init():
        e = e_ref[...]
        eb_ref[...] = e.astype(jnp.bfloat16)
        e2_ref[0, :] = jnp.sum(e * e, axis=0)

    x = x_ref[0]
    x2 = jnp.sum(x * x, axis=0)
    xb2 = (-2.0 * x).astype(jnp.bfloat16)
    acc = jnp.full((M_BLK,), jnp.inf, jnp.float32)
    idx = jnp.zeros((M_BLK,), jnp.int32)
    for c0 in range(0, N_EMBED, N_CHUNK):
        cw = min(N_CHUNK, N_EMBED - c0)
        mm2 = lax.dot_general(xb2, eb_ref[:, c0:c0 + cw],
                              (((0,), (0,)), ((), ())),
                              preferred_element_type=jnp.float32)
        dist = (x2[:, None] + mm2) + e2_ref[0, c0:c0 + cw][None, :]
        m = jnp.min(dist, axis=1, keepdims=True)
        iota = lax.broadcasted_iota(jnp.int32, (M_BLK, cw), 1)
        k = jnp.min(jnp.where(dist == m, iota, cw), axis=1)
        mv = m[:, 0]
        upd = mv < acc
        idx = jnp.where(upd, k + c0, idx)
        acc = jnp.where(upd, mv.astype(jnp.bfloat16).astype(jnp.float32), acc)
    out_ref[0, 0, :] = idx


def _argmin_indices(x3, embed):
    n_blk = x3.shape[0] * HW // M_BLK
    per_b = HW // M_BLK
    out = pl.pallas_call(
        _argmin_body,
        grid=(n_blk,),
        in_specs=[
            pl.BlockSpec((1, DIM, M_BLK), lambda i: (i // per_b, 0, i % per_b)),
            pl.BlockSpec((DIM, N_EMBED), lambda i: (0, 0)),
        ],
        out_specs=pl.BlockSpec((1, 1, M_BLK), lambda i: (i, 0, 0)),
        out_shape=jax.ShapeDtypeStruct((n_blk, 1, M_BLK), jnp.int32),
        scratch_shapes=[
            pltpu.VMEM((DIM, N_EMBED), jnp.bfloat16),
            pltpu.VMEM((1, N_EMBED), jnp.float32),
        ],
    )(x3, embed)
    return out.reshape(x3.shape[0] * HW)


@functools.cache
def _make_gather(nrows):
    info = plsc.get_sparse_core_info()
    nw = info.num_cores * info.num_subcores
    rows_per_w = nrows // nw
    chunk = 128
    n_chunks = rows_per_w // chunk
    mesh = plsc.VectorSubcoreMesh(core_axis_name="c", subcore_axis_name="s")

    @functools.partial(
        pl.kernel, mesh=mesh,
        out_type=jax.ShapeDtypeStruct((nrows, DIM), jnp.float32),
        scratch_types=[
            pltpu.VMEM((n_chunks * chunk,), jnp.int32),
            pltpu.VMEM((chunk, DIM), jnp.float32),
            pltpu.VMEM((chunk, DIM), jnp.float32),
            pltpu.SemaphoreType.DMA,
            pltpu.SemaphoreType.DMA,
        ],
    )
    def gather(table_hbm, idx_hbm, out_hbm, idx_v, rows_a, rows_b, sem_a, sem_b):
        wid = lax.axis_index("s") * info.num_cores + lax.axis_index("c")
        base = wid * rows_per_w
        pltpu.sync_copy(idx_hbm.at[pl.ds(base, rows_per_w)], idx_v)
        bufs = ((rows_a, sem_a), (rows_b, sem_b))
        copies = []
        for i in range(n_chunks):
            rows_v, sem = bufs[i % 2]
            if len(copies) >= 2:
                copies[i - 2].wait()
                pltpu.sync_copy(rows_v, out_hbm.at[pl.ds(base + (i - 2) * chunk, chunk)])
            copies.append(
                pltpu.async_copy(table_hbm.at[idx_v.at[pl.ds(i * chunk, chunk)]],
                                 rows_v, sem))
        for i in range(n_chunks - 2, n_chunks):
            rows_v, _ = bufs[i % 2]
            copies[i].wait()
            pltpu.sync_copy(rows_v, out_hbm.at[pl.ds(base + i * chunk, chunk)])

    return gather


def _transpose_body(q_ref, out1_ref, out2_ref):
    qt = q_ref[0].T
    out1_ref[0] = qt
    out2_ref[0] = qt


def _transpose_dual(qflat):
    q3 = qflat.reshape(B, HW, DIM)
    out = pl.pallas_call(
        _transpose_body,
        grid=(B,),
        in_specs=[pl.BlockSpec((1, HW, DIM), lambda i: (i, 0, 0))],
        out_specs=[pl.BlockSpec((1, DIM, HW), lambda i: (i, 0, 0)),
                   pl.BlockSpec((1, DIM, HW), lambda i: (i, 0, 0))],
        out_shape=[jax.ShapeDtypeStruct((B, DIM, HW), jnp.float32),
                   jax.ShapeDtypeStruct((B, DIM, HW), jnp.float32)],
    )(q3)
    return out[0].reshape(B, DIM, 32, 32), out[1].reshape(B, DIM, 32, 32)


def kernel(input, embed):
    x3 = input.reshape(B, DIM, HW)
    table = embed.T
    hb = B // 2
    gather = _make_gather(hb * HW)
    qs, inds = [], []
    for h in range(2):
        xh = x3[h * hb:(h + 1) * hb]
        ind = _argmin_indices(xh, embed)
        qflat = gather(table, ind)
        qs.append(qflat.reshape(hb, 32, 32, DIM).transpose(0, 3, 1, 2))
        inds.append(ind.reshape(hb, 32, 32))
    q = jnp.concatenate(qs, axis=0)
    ind3 = jnp.concatenate(inds, axis=0)
    return (q, q, ind3)


# speedup vs baseline: 1.1091x; 1.1091x over previous
import functools

import jax
import jax.numpy as jnp
from jax import lax
from jax.experimental import pallas as pl
from jax.experimental.pallas import tpu as pltpu
from jax.experimental.pallas import tpu_sc as plsc

DIM = 256
N_EMBED = 8192
B = 16
HW = 1024
ROWS = B * HW
M_BLK = 512


N_CHUNK = 2816


def _argmin_body(x_ref, e_ref, out_ref, eb_ref, e2_ref):
    @pl.when(pl.program_id(0) == 0)
    def _init():
        e = e_ref[...]
        eb_ref[...] = e.astype(jnp.bfloat16)
        e2_ref[0, :] = jnp.sum(e * e, axis=0)

    x = x_ref[0]
    x2 = jnp.sum(x * x, axis=0)
    xb2 = (-2.0 * x).astype(jnp.bfloat16)
    acc = jnp.full((M_BLK,), jnp.inf, jnp.float32)
    idx = jnp.zeros((M_BLK,), jnp.int32)
    for c0 in range(0, N_EMBED, N_CHUNK):
        cw = min(N_CHUNK, N_EMBED - c0)
        mm2 = lax.dot_general(xb2, eb_ref[:, c0:c0 + cw],
                              (((0,), (0,)), ((), ())),
                              preferred_element_type=jnp.float32)
        dist = (x2[:, None] + mm2) + e2_ref[0, c0:c0 + cw][None, :]
        m = jnp.min(dist, axis=1, keepdims=True)
        iota = lax.broadcasted_iota(jnp.int32, (M_BLK, cw), 1)
        k = jnp.min(jnp.where(dist == m, iota, cw), axis=1)
        mv = m[:, 0]
        upd = mv < acc
        idx = jnp.where(upd, k + c0, idx)
        acc = jnp.where(upd, mv.astype(jnp.bfloat16).astype(jnp.float32), acc)
    out_ref[0, 0, :] = idx


def _argmin_indices(x3, embed):
    n_blk = x3.shape[0] * HW // M_BLK
    per_b = HW // M_BLK
    out = pl.pallas_call(
        _argmin_body,
        grid=(n_blk,),
        in_specs=[
            pl.BlockSpec((1, DIM, M_BLK), lambda i: (i // per_b, 0, i % per_b)),
            pl.BlockSpec((DIM, N_EMBED), lambda i: (0, 0)),
        ],
        out_specs=pl.BlockSpec((1, 1, M_BLK), lambda i: (i, 0, 0)),
        out_shape=jax.ShapeDtypeStruct((n_blk, 1, M_BLK), jnp.int32),
        scratch_shapes=[
            pltpu.VMEM((DIM, N_EMBED), jnp.bfloat16),
            pltpu.VMEM((1, N_EMBED), jnp.float32),
        ],
    )(x3, embed)
    return out.reshape(x3.shape[0] * HW)


@functools.cache
def _make_gather(nrows):
    info = plsc.get_sparse_core_info()
    nw = info.num_cores * info.num_subcores
    rows_per_w = nrows // nw
    chunk = 128
    n_chunks = rows_per_w // chunk
    mesh = plsc.VectorSubcoreMesh(core_axis_name="c", subcore_axis_name="s")

    @functools.partial(
        pl.kernel, mesh=mesh,
        out_type=jax.ShapeDtypeStruct((nrows, DIM), jnp.float32),
        scratch_types=[
            pltpu.VMEM((n_chunks * chunk,), jnp.int32),
            pltpu.VMEM((chunk, DIM), jnp.float32),
            pltpu.VMEM((chunk, DIM), jnp.float32),
            pltpu.SemaphoreType.DMA,
            pltpu.SemaphoreType.DMA,
        ],
    )
    def gather(table_hbm, idx_hbm, out_hbm, idx_v, rows_a, rows_b, sem_a, sem_b):
        wid = lax.axis_index("s") * info.num_cores + lax.axis_index("c")
        base = wid * rows_per_w
        pltpu.sync_copy(idx_hbm.at[pl.ds(base, rows_per_w)], idx_v)
        bufs = ((rows_a, sem_a), (rows_b, sem_b))
        copies = []
        for i in range(n_chunks):
            rows_v, sem = bufs[i % 2]
            if len(copies) >= 2:
                copies[i - 2].wait()
                pltpu.sync_copy(rows_v, out_hbm.at[pl.ds(base + (i - 2) * chunk, chunk)])
            copies.append(
                pltpu.async_copy(table_hbm.at[idx_v.at[pl.ds(i * chunk, chunk)]],
                                 rows_v, sem))
        for i in range(n_chunks - 2, n_chunks):
            rows_v, _ = bufs[i % 2]
            copies[i].wait()
            pltpu.sync_copy(rows_v, out_hbm.at[pl.ds(base + i * chunk, chunk)])

    return gather


def _transpose_body(q_ref, out1_ref, out2_ref):
    qt = q_ref[0].T
    out1_ref[0] = qt
    out2_ref[0] = qt


def _transpose_dual(qflat):
    q3 = qflat.reshape(B, HW, DIM)
    out = pl.pallas_call(
        _transpose_body,
        grid=(B,),
        in_specs=[pl.BlockSpec((1, HW, DIM), lambda i: (i, 0, 0))],
        out_specs=[pl.BlockSpec((1, DIM, HW), lambda i: (i, 0, 0)),
                   pl.BlockSpec((1, DIM, HW), lambda i: (i, 0, 0))],
        out_shape=[jax.ShapeDtypeStruct((B, DIM, HW), jnp.float32),
                   jax.ShapeDtypeStruct((B, DIM, HW), jnp.float32)],
    )(q3)
    return out[0].reshape(B, DIM, 32, 32), out[1].reshape(B, DIM, 32, 32)


def kernel(input, embed):
    x3 = input.reshape(B, DIM, HW)
    table = embed.T
    ind = _argmin_indices(x3, embed)
    qflat = _make_gather(ROWS)(table, ind)
    q = qflat.reshape(B, 32, 32, DIM).transpose(0, 3, 1, 2)
    ind3 = ind.reshape(B, 32, 32)
    return (q, q, ind3)


# speedup vs baseline: 1.1470x; 1.0342x over previous
import functools

import jax
import jax.numpy as jnp
from jax import lax
from jax.experimental import pallas as pl
from jax.experimental.pallas import tpu as pltpu
from jax.experimental.pallas import tpu_sc as plsc

DIM = 256
N_EMBED = 8192
B = 16
HW = 1024
ROWS = B * HW
M_BLK = 1024


N_CHUNK = 2816


def _argmin_body(x_ref, e_ref, out_ref, eb_ref, e2_ref):
    @pl.when(pl.program_id(0) == 0)
    def _init():
        e = e_ref[...]
        eb_ref[...] = e.astype(jnp.bfloat16)
        e2_ref[0, :] = jnp.sum(e * e, axis=0)

    x = x_ref[0]
    x2 = jnp.sum(x * x, axis=0)
    xb2 = (-2.0 * x).astype(jnp.bfloat16)
    acc = jnp.full((M_BLK,), jnp.inf, jnp.float32)
    idx = jnp.zeros((M_BLK,), jnp.int32)
    for c0 in range(0, N_EMBED, N_CHUNK):
        cw = min(N_CHUNK, N_EMBED - c0)
        mm2 = lax.dot_general(xb2, eb_ref[:, c0:c0 + cw],
                              (((0,), (0,)), ((), ())),
                              preferred_element_type=jnp.float32)
        dist = (x2[:, None] + mm2) + e2_ref[0, c0:c0 + cw][None, :]
        m = jnp.min(dist, axis=1, keepdims=True)
        iota = lax.broadcasted_iota(jnp.int32, (M_BLK, cw), 1)
        k = jnp.min(jnp.where(dist == m, iota, cw), axis=1)
        mv = m[:, 0]
        upd = mv < acc
        idx = jnp.where(upd, k + c0, idx)
        acc = jnp.where(upd, mv.astype(jnp.bfloat16).astype(jnp.float32), acc)
    out_ref[0, 0, :] = idx


def _argmin_indices(x3, embed):
    n_blk = x3.shape[0] * HW // M_BLK
    per_b = HW // M_BLK
    out = pl.pallas_call(
        _argmin_body,
        grid=(n_blk,),
        in_specs=[
            pl.BlockSpec((1, DIM, M_BLK), lambda i: (i // per_b, 0, i % per_b)),
            pl.BlockSpec((DIM, N_EMBED), lambda i: (0, 0)),
        ],
        out_specs=pl.BlockSpec((1, 1, M_BLK), lambda i: (i, 0, 0)),
        out_shape=jax.ShapeDtypeStruct((n_blk, 1, M_BLK), jnp.int32),
        scratch_shapes=[
            pltpu.VMEM((DIM, N_EMBED), jnp.bfloat16),
            pltpu.VMEM((1, N_EMBED), jnp.float32),
        ],
    )(x3, embed)
    return out.reshape(x3.shape[0] * HW)


@functools.cache
def _make_gather(nrows):
    info = plsc.get_sparse_core_info()
    nw = info.num_cores * info.num_subcores
    rows_per_w = nrows // nw
    chunk = 128
    n_chunks = rows_per_w // chunk
    mesh = plsc.VectorSubcoreMesh(core_axis_name="c", subcore_axis_name="s")

    @functools.partial(
        pl.kernel, mesh=mesh,
        out_type=jax.ShapeDtypeStruct((nrows, DIM), jnp.float32),
        scratch_types=[
            pltpu.VMEM((n_chunks * chunk,), jnp.int32),
            pltpu.VMEM((chunk, DIM), jnp.float32),
            pltpu.VMEM((chunk, DIM), jnp.float32),
            pltpu.SemaphoreType.DMA,
            pltpu.SemaphoreType.DMA,
        ],
    )
    def gather(table_hbm, idx_hbm, out_hbm, idx_v, rows_a, rows_b, sem_a, sem_b):
        wid = lax.axis_index("s") * info.num_cores + lax.axis_index("c")
        base = wid * rows_per_w
        pltpu.sync_copy(idx_hbm.at[pl.ds(base, rows_per_w)], idx_v)
        bufs = ((rows_a, sem_a), (rows_b, sem_b))
        copies = []
        for i in range(n_chunks):
            rows_v, sem = bufs[i % 2]
            if len(copies) >= 2:
                copies[i - 2].wait()
                pltpu.sync_copy(rows_v, out_hbm.at[pl.ds(base + (i - 2) * chunk, chunk)])
            copies.append(
                pltpu.async_copy(table_hbm.at[idx_v.at[pl.ds(i * chunk, chunk)]],
                                 rows_v, sem))
        for i in range(n_chunks - 2, n_chunks):
            rows_v, _ = bufs[i % 2]
            copies[i].wait()
            pltpu.sync_copy(rows_v, out_hbm.at[pl.ds(base + i * chunk, chunk)])

    return gather


def _transpose_body(q_ref, out1_ref, out2_ref):
    qt = q_ref[0].T
    out1_ref[0] = qt
    out2_ref[0] = qt


def _transpose_dual(qflat):
    q3 = qflat.reshape(B, HW, DIM)
    out = pl.pallas_call(
        _transpose_body,
        grid=(B,),
        in_specs=[pl.BlockSpec((1, HW, DIM), lambda i: (i, 0, 0))],
        out_specs=[pl.BlockSpec((1, DIM, HW), lambda i: (i, 0, 0)),
                   pl.BlockSpec((1, DIM, HW), lambda i: (i, 0, 0))],
        out_shape=[jax.ShapeDtypeStruct((B, DIM, HW), jnp.float32),
                   jax.ShapeDtypeStruct((B, DIM, HW), jnp.float32)],
    )(q3)
    return out[0].reshape(B, DIM, 32, 32), out[1].reshape(B, DIM, 32, 32)


def kernel(input, embed):
    x3 = input.reshape(B, DIM, HW)
    table = embed.T
    ind = _argmin_indices(x3, embed)
    qflat = _make_gather(ROWS)(table, ind)
    q = qflat.reshape(B, 32, 32, DIM).transpose(0, 3, 1, 2)
    ind3 = ind.reshape(B, 32, 32)
    return (q, q, ind3)


# speedup vs baseline: 1.1475x; 1.0004x over previous
import functools

import jax
import jax.numpy as jnp
from jax import lax
from jax.experimental import pallas as pl
from jax.experimental.pallas import tpu as pltpu
from jax.experimental.pallas import tpu_sc as plsc

DIM = 256
N_EMBED = 8192
B = 16
HW = 1024
ROWS = B * HW
M_BLK = 1024


N_CHUNK = 2816


def _argmin_body(x_ref, e_ref, out_ref, eb_ref, e2_ref):
    @pl.when(pl.program_id(0) == 0)
    def _init():
        e = e_ref[...]
        eb_ref[...] = e.astype(jnp.bfloat16)
        e2_ref[0, :] = jnp.sum(e * e, axis=0)

    x = x_ref[0]
    x2 = jnp.sum(x * x, axis=0)
    xb2 = (-2.0 * x).astype(jnp.bfloat16)
    acc = jnp.full((M_BLK,), jnp.inf, jnp.float32)
    idx = jnp.zeros((M_BLK,), jnp.int32)
    for c0 in range(0, N_EMBED, N_CHUNK):
        cw = min(N_CHUNK, N_EMBED - c0)
        mm2 = lax.dot_general(xb2, eb_ref[:, c0:c0 + cw],
                              (((0,), (0,)), ((), ())),
                              preferred_element_type=jnp.float32)
        dist = (x2[:, None] + mm2) + e2_ref[0, c0:c0 + cw][None, :]
        m = jnp.min(dist, axis=1, keepdims=True)
        iota = lax.broadcasted_iota(jnp.int32, (M_BLK, cw), 1)
        k = jnp.min(jnp.where(dist == m, iota, cw), axis=1)
        mv = m[:, 0]
        upd = mv < acc
        idx = jnp.where(upd, k + c0, idx)
        acc = jnp.where(upd, mv.astype(jnp.bfloat16).astype(jnp.float32), acc)
    out_ref[0, 0, :] = idx


def _argmin_indices(x3, embed):
    n_blk = x3.shape[0] * HW // M_BLK
    per_b = HW // M_BLK
    out = pl.pallas_call(
        _argmin_body,
        grid=(n_blk,),
        in_specs=[
            pl.BlockSpec((1, DIM, M_BLK), lambda i: (i // per_b, 0, i % per_b)),
            pl.BlockSpec((DIM, N_EMBED), lambda i: (0, 0)),
        ],
        out_specs=pl.BlockSpec((1, 1, M_BLK), lambda i: (i, 0, 0)),
        out_shape=jax.ShapeDtypeStruct((n_blk, 1, M_BLK), jnp.int32),
        scratch_shapes=[
            pltpu.VMEM((DIM, N_EMBED), jnp.bfloat16),
            pltpu.VMEM((1, N_EMBED), jnp.float32),
        ],
    )(x3, embed)
    return out.reshape(x3.shape[0] * HW)


@functools.cache
def _make_gather(nrows):
    info = plsc.get_sparse_core_info()
    nw = info.num_cores * info.num_subcores
    rows_per_w = nrows // nw
    chunk = 128
    n_chunks = rows_per_w // chunk
    mesh = plsc.VectorSubcoreMesh(core_axis_name="c", subcore_axis_name="s")

    @functools.partial(
        pl.kernel, mesh=mesh,
        out_type=jax.ShapeDtypeStruct((nrows, DIM), jnp.float32),
        scratch_types=[
            pltpu.VMEM((n_chunks * chunk,), jnp.int32),
            pltpu.VMEM((chunk, DIM), jnp.float32),
            pltpu.VMEM((chunk, DIM), jnp.float32),
            pltpu.SemaphoreType.DMA,
            pltpu.SemaphoreType.DMA,
        ],
    )
    def gather(table_hbm, idx_hbm, out_hbm, idx_v, rows_a, rows_b, sem_a, sem_b):
        wid = lax.axis_index("s") * info.num_cores + lax.axis_index("c")
        base = wid * rows_per_w
        pltpu.sync_copy(idx_hbm.at[pl.ds(base, rows_per_w)], idx_v)
        bufs = ((rows_a, sem_a), (rows_b, sem_b))
        copies = []
        for i in range(n_chunks):
            rows_v, sem = bufs[i % 2]
            if len(copies) >= 2:
                copies[i - 2].wait()
                pltpu.sync_copy(rows_v, out_hbm.at[pl.ds(base + (i - 2) * chunk, chunk)])
            copies.append(
                pltpu.async_copy(table_hbm.at[idx_v.at[pl.ds(i * chunk, chunk)]],
                                 rows_v, sem))
        for i in range(n_chunks - 2, n_chunks):
            rows_v, _ = bufs[i % 2]
            copies[i].wait()
            pltpu.sync_copy(rows_v, out_hbm.at[pl.ds(base + i * chunk, chunk)])

    return gather


def kernel(input, embed):
    x3 = input.reshape(B, DIM, HW)
    table = embed.T
    ind = _argmin_indices(x3, embed)
    qflat = _make_gather(ROWS)(table, ind)
    q = qflat.reshape(B, 32, 32, DIM).transpose(0, 3, 1, 2)
    ind3 = ind.reshape(B, 32, 32)
    return (q, q, ind3)
